# Initial kernel scaffold; baseline (speedup 1.0000x reference)
import jax
import jax.numpy as jnp
from jax.experimental import pallas as pl


def kernel(coords, box, pairs, q, p, t):
    raise NotImplementedError("write your pallas kernel here")


# speedup vs baseline: 12.3560x; 12.3560x over previous
import functools
import math

import jax
import jax.numpy as jnp
from jax import lax
from jax.experimental import pallas as pl
from jax.experimental.pallas import tpu as pltpu
from jax.experimental.pallas import tpu_sc as plsc

_NC, _NS, _L = 2, 16, 16
_NW = _NC * _NS
_NP = 800000
_PER_W = _NP // _NW
_C = 1000
_NCHUNK = _PER_W // _C
_IW = 125
_NSUB = _C // _IW
_G = (_C + _L - 1) // _L

_CUTOFF = 10.0
_ALPHA = 0.3
_PREF = 2.0 / math.sqrt(math.pi)

_TROWS = [
    "t0 -tx -ty -tz txx txy txz tyy tyz tzz",
    "tx -txx -txy -txz txxx txxy txxz tyyx txyz tzzx",
    "ty -txy -tyy -tyz txxy tyyx txyz tyyy tyyz tzzy",
    "tz -txz -tyz -tzz txxz txyz tzzx tyyz tzzy tzzz",
    "txx -txxx -txxy -txxz txxxx txxxy txxxz txxyy txxyz txxzz",
    "txy -txxy -tyyx -txyz txxxy txxyy txxyz tyyyx tyyxz tzzxy",
    "txz -txxz -txyz -tzzx txxxz txxyz txxzz tyyxz tzzxy tzzzx",
    "tyy -tyyx -tyyy -tyyz txxyy tyyyx tyyxz tyyyy tyyyz tyyzz",
    "tyz -txyz -tyyz -tzzy txxyz tyyxz tzzxy tyyyz tyyzz tzzzy",
    "tzz -tzzx -tzzy -tzzz txxzz tzzxy tzzzx tyyzz tzzzy tzzzz",
]

_COEF = {}
for _a, _row in enumerate(_TROWS):
    for _b, _tok in enumerate(_row.split()):
        _sign = -1.0 if _tok.startswith("-") else 1.0
        _COEF.setdefault(_tok.lstrip("-"), []).append((_sign, _a, _b))


def _rb(x):
    i = lax.bitcast_convert_type(x, jnp.int32)
    r = i + jnp.int32(0x7FFF) + (lax.shift_right_logical(i, 16) & jnp.int32(1))
    r = r & jnp.int32(-65536)
    return lax.bitcast_convert_type(r, jnp.float32)


def _sqrt(x):
    i = lax.bitcast_convert_type(x, jnp.int32)
    i = jnp.int32(0x5F3759DF) - lax.shift_right_logical(i, 1)
    y = lax.bitcast_convert_type(i, jnp.float32)
    xh = 0.5 * x
    for _ in range(3):
        y = y * (1.5 - xh * y * y)
    s = x * y
    return 0.5 * (s + x / s)


def _erfc(u, exp2u):
    t = 1.0 / (1.0 + 0.3275911 * u)
    poly = t * (0.254829592 + t * (-0.284496736 + t * (
        1.421413741 + t * (-1.453152027 + t * 1.061405429))))
    return poly * exp2u


def _pair_energy(dx, dy, dz, mi, mj):
    dr2 = dx * dx + dy * dy + dz * dz
    dr = _sqrt(dr2)
    drInv = 1.0 / dr
    u = _ALPHA * dr
    exp2u = jnp.exp(-u * u)
    erfc_u = _erfc(u, exp2u)
    u2 = u * u
    u3 = u2 * u
    u5 = u3 * u2
    u7 = u5 * u2
    p3 = u
    p5 = (3 * u + 2 * u3) / 3
    p7 = (15 * u + 10 * u3 + 4 * u5) / 15
    p9 = (8 * u7 + 28 * u5 + 70 * u3 + 105 * u) / 105
    f1 = erfc_u
    f3 = erfc_u + _PREF * p3 * exp2u
    f5 = erfc_u + _PREF * p5 * exp2u
    f7 = erfc_u + _PREF * p7 * exp2u
    f9 = erfc_u + _PREF * p9 * exp2u

    drInv2 = drInv * drInv
    drInv3 = drInv2 * drInv
    drInv5 = drInv3 * drInv2
    drInv7 = drInv5 * drInv2
    drInv9 = drInv7 * drInv2
    D1 = drInv * f1
    D3 = drInv3 * f3
    D5 = drInv5 * f5
    D7 = drInv7 * f7
    D9 = drInv9 * f9

    x, y, z = dx, dy, dz
    x2, y2, z2 = x * x, y * y, z * z
    xy, xz, yz = x * y, x * z, y * z

    t = {}
    t["t0"] = D1
    t["tx"] = -x * D3
    t["ty"] = -y * D3
    t["tz"] = -z * D3
    t["txx"] = 3 * x2 * D5 - D3
    t["txy"] = 3 * xy * D5
    t["txz"] = 3 * xz * D5
    t["tyy"] = 3 * y2 * D5 - D3
    t["tyz"] = 3 * yz * D5
    t["tzz"] = 3 * z2 * D5 - D3
    t["txxx"] = -15 * x2 * x * D7 + 9 * x * D5
    t["txxy"] = -15 * x2 * y * D7 + 3 * y * D5
    t["txxz"] = -15 * x2 * z * D7 + 3 * z * D5
    t["tyyy"] = -15 * y2 * y * D7 + 9 * y * D5
    t["tyyx"] = -15 * y2 * x * D7 + 3 * x * D5
    t["tyyz"] = -15 * y2 * z * D7 + 3 * z * D5
    t["tzzz"] = -15 * z2 * z * D7 + 9 * z * D5
    t["tzzx"] = -15 * z2 * x * D7 + 3 * x * D5
    t["tzzy"] = -15 * z2 * y * D7 + 3 * y * D5
    t["txyz"] = -15 * xy * z * D7
    t["txxxx"] = 105 * x2 * x2 * D9 - 90 * x2 * D7 + 9 * D5
    t["txxxy"] = 105 * x2 * xy * D9 - 45 * xy * D7
    t["txxxz"] = 105 * x2 * xz * D9 - 45 * xz * D7
    t["txxyy"] = 105 * x2 * y2 * D9 - 15 * (x2 + y2) * D7 + 3 * D5
    t["txxzz"] = 105 * x2 * z2 * D9 - 15 * (x2 + z2) * D7 + 3 * D5
    t["txxyz"] = 105 * x2 * yz * D9 - 15 * yz * D7
    t["tyyyy"] = 105 * y2 * y2 * D9 - 90 * y2 * D7 + 9 * D5
    t["tyyyx"] = 105 * y2 * xy * D9 - 45 * xy * D7
    t["tyyyz"] = 105 * y2 * yz * D9 - 45 * yz * D7
    t["tyyzz"] = 105 * y2 * z2 * D9 - 15 * (y2 + z2) * D7 + 3 * D5
    t["tyyxz"] = 105 * y2 * xz * D9 - 15 * xz * D7
    t["tzzzz"] = 105 * z2 * z2 * D9 - 90 * z2 * D7 + 9 * D5
    t["tzzzx"] = 105 * z2 * xz * D9 - 45 * xz * D7
    t["tzzzy"] = 105 * z2 * yz * D9 - 45 * yz * D7
    t["tzzxy"] = 105 * z2 * xy * D9 - 15 * xy * D7

    ene = None
    for name, terms in _COEF.items():
        w = None
        for sign, a, b in terms:
            prod = mi[a] * mj[b]
            contrib = prod if sign > 0 else -prod
            w = contrib if w is None else w + contrib
        term = t[name] * w
        ene = term if ene is None else ene + term
    return jnp.where(dr <= _CUTOFF, ene, 0.0)


def _sc_energy_partials(table, srcs2, dsts2, boxv):
    mesh = plsc.VectorSubcoreMesh(
        core_axis_name="c", subcore_axis_name="s",
        num_cores=_NC, num_subcores=_NS)

    @functools.partial(
        pl.kernel,
        mesh=mesh,
        out_type=jax.ShapeDtypeStruct((_NW, _L), jnp.float32),
        scratch_types=[
            pltpu.VMEM((_NSUB, _IW), jnp.int32),
            pltpu.VMEM((_NSUB, _IW), jnp.int32),
            pltpu.VMEM((_C, _L), jnp.float32),
            pltpu.VMEM((_C, _L), jnp.float32),
            pltpu.VMEM((3, _L), jnp.float32),
            pltpu.VMEM((_L,), jnp.float32),
            pltpu.SemaphoreType.DMA,
            pltpu.SemaphoreType.DMA,
        ],
        compiler_params=pltpu.CompilerParams(
            use_tc_tiling_on_sc=False, needs_layout_passes=False),
    )
    def sc_kernel(table_h, srcs_h, dsts_h, boxv_h, out_h,
                  sidx, didx, srows, drows, boxb, accv, sem1, sem2):
        wid = lax.axis_index("s") * _NC + lax.axis_index("c")
        pltpu.sync_copy(boxv_h, boxb)
        bx = _rb(boxb[0, :])
        by = _rb(boxb[1, :])
        bz = _rb(boxb[2, :])
        ibx = _rb(1.0 / boxb[0, :])
        iby = _rb(1.0 / boxb[1, :])
        ibz = _rb(1.0 / boxb[2, :])

        def wrap(d, b, ib):
            s = _rb(d) * ib
            n = (jnp.where(s > 0.5, 1.0, 0.0)
                 - jnp.where(s < -0.5, 1.0, 0.0))
            return _rb(s - n) * b

        def chunk_body(ci, acc):
            cg = wid * _NCHUNK + ci
            pltpu.sync_copy(srcs_h.at[pl.ds(cg * _NSUB, _NSUB)], sidx)
            pltpu.sync_copy(dsts_h.at[pl.ds(cg * _NSUB, _NSUB)], didx)
            cps = []
            for r in range(_NSUB):
                cps.append(pltpu.async_copy(
                    table_h.at[sidx.at[r]],
                    srows.at[pl.ds(r * _IW, _IW)], sem1))
                cps.append(pltpu.async_copy(
                    table_h.at[didx.at[r]],
                    drows.at[pl.ds(r * _IW, _IW)], sem2))
            for cp in cps:
                cp.wait()

            def group_body(g, acc2):
                lanes = g * _L + lax.iota(jnp.int32, _L)
                valid = lanes < _C
                ridx = jnp.minimum(lanes, _C - 1)

                def col(ref, c):
                    cvec = jnp.full((_L,), c, dtype=jnp.int32)
                    return plsc.load_gather(ref, [ridx, cvec])

                xi = col(srows, 0)
                yi = col(srows, 1)
                zi = col(srows, 2)
                xj = col(drows, 0)
                yj = col(drows, 1)
                zj = col(drows, 2)
                mi = [col(srows, 3 + k) for k in range(10)]
                mj = [col(drows, 3 + k) for k in range(10)]
                dx = wrap(xj - xi, bx, ibx)
                dy = wrap(yj - yi, by, iby)
                dz = wrap(zj - zi, bz, ibz)
                ene = _pair_energy(dx, dy, dz, mi, mj)
                return acc2 + jnp.where(valid, ene, 0.0)

            return lax.fori_loop(0, _G, group_body, acc)

        acc = lax.fori_loop(0, _NCHUNK, chunk_body,
                            jnp.zeros((_L,), jnp.float32))
        accv[...] = acc
        pltpu.sync_copy(accv, out_h.at[wid])

    return sc_kernel(table, srcs2, dsts2, boxv)


def _sum_kernel(x_ref, o_ref):
    o_ref[...] = (jnp.sum(x_ref[...]) * 1.0).reshape(1, 1)


def kernel(coords, box, pairs, q, p, t):
    qv = q[:, None]
    packed = jnp.concatenate([
        coords.astype(jnp.float32), qv, p,
        t[:, 0, 0][:, None] / 3,
        (t[:, 0, 1] + t[:, 1, 0])[:, None] / 3,
        (t[:, 0, 2] + t[:, 2, 0])[:, None] / 3,
        t[:, 1, 1][:, None] / 3,
        (t[:, 1, 2] + t[:, 2, 1])[:, None] / 3,
        t[:, 2, 2][:, None] / 3,
        jnp.zeros((coords.shape[0], 3), jnp.float32)], axis=1)
    srcs2 = pairs[:, 0].reshape(_NP // _IW, _IW)
    dsts2 = pairs[:, 1].reshape(_NP // _IW, _IW)
    boxv = jnp.broadcast_to(
        jnp.diagonal(box).astype(jnp.float32)[:, None], (3, _L))
    partials = _sc_energy_partials(packed, srcs2, dsts2, boxv)
    total = pl.pallas_call(
        _sum_kernel,
        out_shape=jax.ShapeDtypeStruct((1, 1), jnp.float32),
    )(partials)
    return total[0, 0]


# speedup vs baseline: 13.4209x; 1.0862x over previous
import functools
import math

import jax
import jax.numpy as jnp
from jax import lax
from jax.experimental import pallas as pl
from jax.experimental.pallas import tpu as pltpu
from jax.experimental.pallas import tpu_sc as plsc

_NC, _NS, _L = 2, 16, 16
_NW = _NC * _NS
_NP = 800000
_PER_W = _NP // _NW
_C = 625
_NCHUNK = _PER_W // _C
_IW = 125
_NSUB = _C // _IW
_G = (_C + _L - 1) // _L

_CUTOFF = 10.0
_ALPHA = 0.3
_PREF = 2.0 / math.sqrt(math.pi)

_TROWS = [
    "t0 -tx -ty -tz txx txy txz tyy tyz tzz",
    "tx -txx -txy -txz txxx txxy txxz tyyx txyz tzzx",
    "ty -txy -tyy -tyz txxy tyyx txyz tyyy tyyz tzzy",
    "tz -txz -tyz -tzz txxz txyz tzzx tyyz tzzy tzzz",
    "txx -txxx -txxy -txxz txxxx txxxy txxxz txxyy txxyz txxzz",
    "txy -txxy -tyyx -txyz txxxy txxyy txxyz tyyyx tyyxz tzzxy",
    "txz -txxz -txyz -tzzx txxxz txxyz txxzz tyyxz tzzxy tzzzx",
    "tyy -tyyx -tyyy -tyyz txxyy tyyyx tyyxz tyyyy tyyyz tyyzz",
    "tyz -txyz -tyyz -tzzy txxyz tyyxz tzzxy tyyyz tyyzz tzzzy",
    "tzz -tzzx -tzzy -tzzz txxzz tzzxy tzzzx tyyzz tzzzy tzzzz",
]

_COEF = {}
for _a, _row in enumerate(_TROWS):
    for _b, _tok in enumerate(_row.split()):
        _sign = -1.0 if _tok.startswith("-") else 1.0
        _COEF.setdefault(_tok.lstrip("-"), []).append((_sign, _a, _b))


def _rb(x):
    i = lax.bitcast_convert_type(x, jnp.int32)
    r = i + jnp.int32(0x7FFF) + (lax.shift_right_logical(i, 16) & jnp.int32(1))
    r = r & jnp.int32(-65536)
    return lax.bitcast_convert_type(r, jnp.float32)


def _sqrt(x):
    i = lax.bitcast_convert_type(x, jnp.int32)
    i = jnp.int32(0x5F3759DF) - lax.shift_right_logical(i, 1)
    y = lax.bitcast_convert_type(i, jnp.float32)
    xh = 0.5 * x
    for _ in range(3):
        y = y * (1.5 - xh * y * y)
    s = x * y
    return 0.5 * (s + x / s)


def _erfc(u, exp2u):
    t = 1.0 / (1.0 + 0.3275911 * u)
    poly = t * (0.254829592 + t * (-0.284496736 + t * (
        1.421413741 + t * (-1.453152027 + t * 1.061405429))))
    return poly * exp2u


def _pair_energy(dx, dy, dz, mi, mj):
    dr2 = dx * dx + dy * dy + dz * dz
    dr = _sqrt(dr2)
    drInv = 1.0 / dr
    u = _ALPHA * dr
    exp2u = jnp.exp(-u * u)
    erfc_u = _erfc(u, exp2u)
    u2 = u * u
    u3 = u2 * u
    u5 = u3 * u2
    u7 = u5 * u2
    p3 = u
    p5 = (3 * u + 2 * u3) / 3
    p7 = (15 * u + 10 * u3 + 4 * u5) / 15
    p9 = (8 * u7 + 28 * u5 + 70 * u3 + 105 * u) / 105
    f1 = erfc_u
    f3 = erfc_u + _PREF * p3 * exp2u
    f5 = erfc_u + _PREF * p5 * exp2u
    f7 = erfc_u + _PREF * p7 * exp2u
    f9 = erfc_u + _PREF * p9 * exp2u

    drInv2 = drInv * drInv
    drInv3 = drInv2 * drInv
    drInv5 = drInv3 * drInv2
    drInv7 = drInv5 * drInv2
    drInv9 = drInv7 * drInv2
    D1 = drInv * f1
    D3 = drInv3 * f3
    D5 = drInv5 * f5
    D7 = drInv7 * f7
    D9 = drInv9 * f9

    x, y, z = dx, dy, dz
    x2, y2, z2 = x * x, y * y, z * z
    xy, xz, yz = x * y, x * z, y * z

    t = {}
    t["t0"] = D1
    t["tx"] = -x * D3
    t["ty"] = -y * D3
    t["tz"] = -z * D3
    t["txx"] = 3 * x2 * D5 - D3
    t["txy"] = 3 * xy * D5
    t["txz"] = 3 * xz * D5
    t["tyy"] = 3 * y2 * D5 - D3
    t["tyz"] = 3 * yz * D5
    t["tzz"] = 3 * z2 * D5 - D3
    t["txxx"] = -15 * x2 * x * D7 + 9 * x * D5
    t["txxy"] = -15 * x2 * y * D7 + 3 * y * D5
    t["txxz"] = -15 * x2 * z * D7 + 3 * z * D5
    t["tyyy"] = -15 * y2 * y * D7 + 9 * y * D5
    t["tyyx"] = -15 * y2 * x * D7 + 3 * x * D5
    t["tyyz"] = -15 * y2 * z * D7 + 3 * z * D5
    t["tzzz"] = -15 * z2 * z * D7 + 9 * z * D5
    t["tzzx"] = -15 * z2 * x * D7 + 3 * x * D5
    t["tzzy"] = -15 * z2 * y * D7 + 3 * y * D5
    t["txyz"] = -15 * xy * z * D7
    t["txxxx"] = 105 * x2 * x2 * D9 - 90 * x2 * D7 + 9 * D5
    t["txxxy"] = 105 * x2 * xy * D9 - 45 * xy * D7
    t["txxxz"] = 105 * x2 * xz * D9 - 45 * xz * D7
    t["txxyy"] = 105 * x2 * y2 * D9 - 15 * (x2 + y2) * D7 + 3 * D5
    t["txxzz"] = 105 * x2 * z2 * D9 - 15 * (x2 + z2) * D7 + 3 * D5
    t["txxyz"] = 105 * x2 * yz * D9 - 15 * yz * D7
    t["tyyyy"] = 105 * y2 * y2 * D9 - 90 * y2 * D7 + 9 * D5
    t["tyyyx"] = 105 * y2 * xy * D9 - 45 * xy * D7
    t["tyyyz"] = 105 * y2 * yz * D9 - 45 * yz * D7
    t["tyyzz"] = 105 * y2 * z2 * D9 - 15 * (y2 + z2) * D7 + 3 * D5
    t["tyyxz"] = 105 * y2 * xz * D9 - 15 * xz * D7
    t["tzzzz"] = 105 * z2 * z2 * D9 - 90 * z2 * D7 + 9 * D5
    t["tzzzx"] = 105 * z2 * xz * D9 - 45 * xz * D7
    t["tzzzy"] = 105 * z2 * yz * D9 - 45 * yz * D7
    t["tzzxy"] = 105 * z2 * xy * D9 - 15 * xy * D7

    ene = None
    for name, terms in _COEF.items():
        w = None
        for sign, a, b in terms:
            prod = mi[a] * mj[b]
            contrib = prod if sign > 0 else -prod
            w = contrib if w is None else w + contrib
        term = t[name] * w
        ene = term if ene is None else ene + term
    return jnp.where(dr <= _CUTOFF, ene, 0.0)


def _sc_energy_partials(table, sd2, boxv):
    mesh = plsc.VectorSubcoreMesh(
        core_axis_name="c", subcore_axis_name="s",
        num_cores=_NC, num_subcores=_NS)

    @functools.partial(
        pl.kernel,
        mesh=mesh,
        out_type=jax.ShapeDtypeStruct((_NW, _L), jnp.float32),
        scratch_types=[
            pltpu.VMEM((2, _NSUB, _IW), jnp.int32),
            pltpu.VMEM((2, _NSUB, _IW), jnp.int32),
            pltpu.VMEM((_C, _L), jnp.float32),
            pltpu.VMEM((_C, _L), jnp.float32),
            pltpu.VMEM((_C, _L), jnp.float32),
            pltpu.VMEM((_C, _L), jnp.float32),
            pltpu.VMEM((3, _L), jnp.float32),
            pltpu.VMEM((_L,), jnp.float32),
            pltpu.SemaphoreType.DMA,
            pltpu.SemaphoreType.DMA,
        ],
        compiler_params=pltpu.CompilerParams(
            use_tc_tiling_on_sc=False, needs_layout_passes=False),
    )
    def sc_kernel(table_h, sd_h, boxv_h, out_h,
                  idxA, idxB, srA, drA, srB, drB, boxb, accv, semA, semB):
        wid = lax.axis_index("s") * _NC + lax.axis_index("c")
        pltpu.sync_copy(boxv_h, boxb)
        bx = _rb(boxb[0, :])
        by = _rb(boxb[1, :])
        bz = _rb(boxb[2, :])
        ibx = _rb(1.0 / boxb[0, :])
        iby = _rb(1.0 / boxb[1, :])
        ibz = _rb(1.0 / boxb[2, :])

        def wrap(d, b, ib):
            s = _rb(d) * ib
            n = (jnp.where(s > 0.5, 1.0, 0.0)
                 - jnp.where(s < -0.5, 1.0, 0.0))
            return _rb(s - n) * b

        def load_idx(cg, idxbuf):
            pltpu.sync_copy(sd_h.at[:, pl.ds(cg * _NSUB, _NSUB)], idxbuf)

        def issue(idxbuf, sr, dr, sem):
            for r in range(_NSUB):
                pltpu.async_copy(table_h.at[idxbuf.at[0, r]],
                                 sr.at[pl.ds(r * _IW, _IW)], sem)
                pltpu.async_copy(table_h.at[idxbuf.at[1, r]],
                                 dr.at[pl.ds(r * _IW, _IW)], sem)

        def drain(idxbuf, sr, dr, sem):
            for r in range(_NSUB):
                pltpu.make_async_copy(table_h.at[idxbuf.at[0, r]],
                                      sr.at[pl.ds(r * _IW, _IW)], sem).wait()
                pltpu.make_async_copy(table_h.at[idxbuf.at[1, r]],
                                      dr.at[pl.ds(r * _IW, _IW)], sem).wait()

        def compute(srows, drows, acc):
            def group_body(g, acc2):
                lanes = g * _L + lax.iota(jnp.int32, _L)
                valid = lanes < _C
                ridx = jnp.minimum(lanes, _C - 1)

                def col(ref, c):
                    cvec = jnp.full((_L,), c, dtype=jnp.int32)
                    return plsc.load_gather(ref, [ridx, cvec])

                xi = col(srows, 0)
                yi = col(srows, 1)
                zi = col(srows, 2)
                xj = col(drows, 0)
                yj = col(drows, 1)
                zj = col(drows, 2)
                mi = [col(srows, 3 + k) for k in range(10)]
                mj = [col(drows, 3 + k) for k in range(10)]
                dx = wrap(xj - xi, bx, ibx)
                dy = wrap(yj - yi, by, iby)
                dz = wrap(zj - zi, bz, ibz)
                ene = _pair_energy(dx, dy, dz, mi, mj)
                return acc2 + jnp.where(valid, ene, 0.0)

            return lax.fori_loop(0, _G, group_body, acc)

        cg0 = wid * _NCHUNK
        load_idx(cg0, idxA)
        issue(idxA, srA, drA, semA)

        def body(k, acc):
            c0 = cg0 + 2 * k
            load_idx(c0 + 1, idxB)
            drain(idxA, srA, drA, semA)
            issue(idxB, srB, drB, semB)
            acc = compute(srA, drA, acc)

            @pl.when(k < _NCHUNK // 2 - 1)
            def _():
                load_idx(c0 + 2, idxA)
                issue(idxA, srA, drA, semA)

            drain(idxB, srB, drB, semB)
            return compute(srB, drB, acc)

        acc = lax.fori_loop(0, _NCHUNK // 2, body,
                            jnp.zeros((_L,), jnp.float32))
        accv[...] = acc
        pltpu.sync_copy(accv, out_h.at[wid])

    return sc_kernel(table, sd2, boxv)


def _sum_kernel(x_ref, o_ref):
    o_ref[...] = (jnp.sum(x_ref[...]) * 1.0).reshape(1, 1)


def kernel(coords, box, pairs, q, p, t):
    qv = q[:, None]
    packed = jnp.concatenate([
        coords.astype(jnp.float32), qv, p,
        t[:, 0, 0][:, None] / 3,
        (t[:, 0, 1] + t[:, 1, 0])[:, None] / 3,
        (t[:, 0, 2] + t[:, 2, 0])[:, None] / 3,
        t[:, 1, 1][:, None] / 3,
        (t[:, 1, 2] + t[:, 2, 1])[:, None] / 3,
        t[:, 2, 2][:, None] / 3,
        jnp.zeros((coords.shape[0], 3), jnp.float32)], axis=1)
    sd2 = jnp.stack([pairs[:, 0].reshape(_NP // _IW, _IW),
                     pairs[:, 1].reshape(_NP // _IW, _IW)], axis=0)
    boxv = jnp.broadcast_to(
        jnp.diagonal(box).astype(jnp.float32)[:, None], (3, _L))
    partials = _sc_energy_partials(packed, sd2, boxv)
    total = pl.pallas_call(
        _sum_kernel,
        out_shape=jax.ShapeDtypeStruct((1, 1), jnp.float32),
    )(partials)
    return total[0, 0]
